# Initial kernel scaffold; baseline (speedup 1.0000x reference)
import jax
import jax.numpy as jnp
from jax.experimental import pallas as pl


def kernel(x, embedding):
    raise NotImplementedError("write your pallas kernel here")


# speedup vs baseline: 1.0232x; 1.0232x over previous
import functools

import jax
import jax.numpy as jnp
from jax import lax
from jax.experimental import pallas as pl
from jax.experimental.pallas import tpu as pltpu
from jax.experimental.pallas import tpu_sc as plsc


def _gather_rows(n_rows, n_workers, chunk, d):
    b_per_w = n_rows // n_workers
    n_chunks = b_per_w // chunk
    assert n_chunks % 2 == 0 and chunk % 8 == 0
    mesh = plsc.VectorSubcoreMesh(core_axis_name="c", subcore_axis_name="s")

    @functools.partial(
        pl.kernel,
        mesh=mesh,
        compiler_params=pltpu.CompilerParams(use_tc_tiling_on_sc=False),
        out_type=jax.ShapeDtypeStruct((n_rows, d), jnp.float32),
        scratch_types=[
            pltpu.VMEM((b_per_w,), jnp.int32),
            pltpu.VMEM((chunk, d), jnp.float32),
            pltpu.VMEM((chunk, d), jnp.float32),
            pltpu.SemaphoreType.DMA,
            pltpu.SemaphoreType.DMA,
            pltpu.SemaphoreType.DMA,
            pltpu.SemaphoreType.DMA,
        ],
    )
    def k(idx_hbm, table_hbm, out_hbm, idx_v, rows0, rows1,
          gsem0, gsem1, ssem0, ssem1):
        nc = lax.axis_size("c")
        wid = lax.axis_index("s") * nc + lax.axis_index("c")
        base = wid * b_per_w
        pltpu.sync_copy(idx_hbm.at[pl.ds(base, b_per_w)], idx_v)

        def gather(i, buf, sem):
            return pltpu.async_copy(
                table_hbm.at[idx_v.at[pl.ds(i * chunk, chunk)]], buf, sem
            )

        def scatter(i, buf, sem):
            return pltpu.async_copy(
                buf, out_hbm.at[pl.ds(base + i * chunk, chunk)], sem
            )

        @pl.loop(0, n_chunks, step=2)
        def body(i0):
            g0 = gather(i0, rows0, gsem0)
            g1 = gather(i0 + 1, rows1, gsem1)
            g0.wait()
            s0 = scatter(i0, rows0, ssem0)
            g1.wait()
            s1 = scatter(i0 + 1, rows1, ssem1)
            s0.wait()
            s1.wait()

    return k


def kernel(x, embedding):
    b, s = x.shape
    v, d = embedding.shape
    n = b * s
    idx = x.reshape(n).astype(jnp.int32)
    out = _gather_rows(n, 32, 32, d)(idx, embedding)
    return out.reshape(b, s, d)


# speedup vs baseline: 1.0267x; 1.0034x over previous
import functools

import jax
import jax.numpy as jnp
from jax import lax
from jax.experimental import pallas as pl
from jax.experimental.pallas import tpu as pltpu
from jax.experimental.pallas import tpu_sc as plsc

_NBUF = 4


def _gather_rows(n_rows, n_workers, chunk, d):
    b_per_w = n_rows // n_workers
    n_chunks = b_per_w // chunk
    assert b_per_w % chunk == 0 and chunk % 8 == 0
    assert n_chunks % _NBUF == 0 and n_chunks >= 2 * _NBUF
    mesh = plsc.VectorSubcoreMesh(core_axis_name="c", subcore_axis_name="s")

    @functools.partial(
        pl.kernel,
        mesh=mesh,
        compiler_params=pltpu.CompilerParams(use_tc_tiling_on_sc=False),
        out_type=jax.ShapeDtypeStruct((n_rows, d), jnp.float32),
        scratch_types=[
            pltpu.VMEM((b_per_w,), jnp.int32),
            [pltpu.VMEM((chunk, d), jnp.float32)] * _NBUF,
            [pltpu.SemaphoreType.DMA] * _NBUF,
            [pltpu.SemaphoreType.DMA] * _NBUF,
        ],
    )
    def k(idx_hbm, table_hbm, out_hbm, idx_v, bufs, gsems, ssems):
        nc = lax.axis_size("c")
        wid = lax.axis_index("s") * nc + lax.axis_index("c")
        base = wid * b_per_w
        pltpu.sync_copy(idx_hbm.at[pl.ds(base, b_per_w)], idx_v)

        def gather(i, b):
            pltpu.async_copy(
                table_hbm.at[idx_v.at[pl.ds(i * chunk, chunk)]],
                bufs[b], gsems[b],
            )

        def scatter(i, b):
            pltpu.async_copy(
                bufs[b], out_hbm.at[pl.ds(base + i * chunk, chunk)], ssems[b]
            )

        def gwait(b):
            pltpu.make_async_copy(
                table_hbm.at[pl.ds(0, chunk)], bufs[b], gsems[b]
            ).wait()

        def swait(b):
            pltpu.make_async_copy(
                bufs[b], out_hbm.at[pl.ds(base, chunk)], ssems[b]
            ).wait()

        for b in range(_NBUF):
            gather(b, b)

        @pl.loop(_NBUF, n_chunks, step=_NBUF)
        def body(i0):
            for b in range(_NBUF):
                gwait(b)
                scatter(i0 - _NBUF + b, b)
            for b in range(_NBUF):
                swait(b)
                gather(i0 + b, b)

        for b in range(_NBUF):
            gwait(b)
            scatter(n_chunks - _NBUF + b, b)
        for b in range(_NBUF):
            swait(b)

    return k


def kernel(x, embedding):
    b, s = x.shape
    v, d = embedding.shape
    n = b * s
    idx = x.reshape(n).astype(jnp.int32)
    out = _gather_rows(n, 32, 16, d)(idx, embedding)
    return out.reshape(b, s, d)
